# baseline (speedup 1.0000x reference)
import functools

import jax
import jax.numpy as jnp
from jax import lax
from jax.experimental import pallas as pl
from jax.experimental.pallas import tpu as pltpu
from jax.experimental.pallas import tpu_sc as plsc

_NUM_CORES = 2
_NUM_SUBCORES = 16
_NUM_WORKERS = _NUM_CORES * _NUM_SUBCORES
_LANES = 16

_NUM_EVENTS = 1969
_BATCH = 16384
_D_OUT = 5
_B_PER_W = _BATCH // _NUM_WORKERS
_E_PER_W = _B_PER_W * _D_OUT


def _project_body(ev_ref, w_ref, b_ref, out_ref):
    out_ref[...] = (
        jnp.dot(ev_ref[...], w_ref[...], preferred_element_type=jnp.float32)
        + b_ref[...]
    )


def _project(events, w, b2d):
    return pl.pallas_call(
        _project_body,
        out_shape=jax.ShapeDtypeStruct((_NUM_EVENTS, _D_OUT), jnp.float32),
    )(events, w, b2d)


_sc_mesh = plsc.VectorSubcoreMesh(
    core_axis_name="c",
    subcore_axis_name="s",
    num_cores=_NUM_CORES,
    num_subcores=_NUM_SUBCORES,
)


@functools.partial(
    pl.kernel,
    out_type=jax.ShapeDtypeStruct((_BATCH * _D_OUT,), jnp.float32),
    mesh=_sc_mesh,
    scratch_types=[
        pltpu.VMEM((_B_PER_W,), jnp.int32),
        pltpu.VMEM((_E_PER_W,), jnp.int32),
        pltpu.VMEM((_E_PER_W,), jnp.float32),
        pltpu.SemaphoreType.DMA,
    ],
    compiler_params=pltpu.CompilerParams(
        use_tc_tiling_on_sc=False, needs_layout_passes=False
    ),
)
def _gather_elems(table_hbm, idx_hbm, out_hbm, idx_v, eidx_v, vals_v, sem):
    wid = lax.axis_index("s") * _NUM_CORES + lax.axis_index("c")
    base = wid * _B_PER_W
    pltpu.sync_copy(idx_hbm.at[pl.ds(base, _B_PER_W)], idx_v)
    pos0 = jnp.arange(_LANES, dtype=jnp.int32) * _D_OUT
    for g in range(_B_PER_W // _LANES):
        d5 = idx_v[pl.ds(g * _LANES, _LANES)] * _D_OUT
        pos = pos0 + (g * _LANES * _D_OUT)
        for c in range(_D_OUT):
            plsc.store_scatter(eidx_v, [pos + c], d5 + c)
    pltpu.async_copy(table_hbm.at[eidx_v], vals_v, sem).wait()
    pltpu.sync_copy(vals_v, out_hbm.at[pl.ds(base * _D_OUT, _E_PER_W)])


def kernel(days_index, events, W, b):
    table = _project(events, W, b.reshape(1, _D_OUT))
    flat = _gather_elems(table.reshape(-1), days_index)
    return flat.reshape(_BATCH, _D_OUT)
